# baseline (speedup 1.0000x reference)
import jax
import jax.numpy as jnp
from jax.experimental import pallas as pl
from jax.experimental.pallas import tpu as pltpu

B, I, F = 4096, 16, 512
BB = 128
GRID = B // BB


def _loss_kernel(out_ref, lab_ref, imp_ref, o_ref, acc_ref):
    step = pl.program_id(0)

    d = imp_ref[...] * (jnp.abs(lab_ref[...]) - out_ref[...])
    partial = jnp.sum(d * d, axis=0)

    @pl.when(step == 0)
    def _():
        acc_ref[...] = partial

    @pl.when(step > 0)
    def _():
        acc_ref[...] = acc_ref[...] + partial

    @pl.when(step == GRID - 1)
    def _():
        o_ref[0, :] = jnp.sum(acc_ref[...], axis=1) * (1.0 / (B * F))


def kernel(out, labels, importance):
    res = pl.pallas_call(
        _loss_kernel,
        grid=(GRID,),
        in_specs=[
            pl.BlockSpec((BB, I, F), lambda g: (g, 0, 0)),
            pl.BlockSpec((BB, I, F), lambda g: (g, 0, 0)),
            pl.BlockSpec((I, F), lambda g: (0, 0)),
        ],
        out_specs=pl.BlockSpec((1, I), lambda g: (0, 0)),
        out_shape=jax.ShapeDtypeStruct((1, I), jnp.float32),
        scratch_shapes=[pltpu.VMEM((I, F), jnp.float32)],
    )(out, labels, importance)
    return res[0]


# speedup vs baseline: 1.0504x; 1.0504x over previous
import jax
import jax.numpy as jnp
from jax.experimental import pallas as pl
from jax.experimental.pallas import tpu as pltpu

B, I, F = 4096, 16, 512
BB = 256
GRID = B // BB


def _loss_kernel(out_ref, lab_ref, imp_ref, o_ref, acc_ref):
    step = pl.program_id(0)

    d = imp_ref[...] * (jnp.abs(lab_ref[...]) - out_ref[...])
    partial = jnp.sum(d * d, axis=0)

    @pl.when(step == 0)
    def _():
        acc_ref[...] = partial

    @pl.when(step > 0)
    def _():
        acc_ref[...] = acc_ref[...] + partial

    @pl.when(step == GRID - 1)
    def _():
        o_ref[0, :] = jnp.sum(acc_ref[...], axis=1) * (1.0 / (B * F))


def kernel(out, labels, importance):
    res = pl.pallas_call(
        _loss_kernel,
        grid=(GRID,),
        in_specs=[
            pl.BlockSpec((BB, I, F), lambda g: (g, 0, 0)),
            pl.BlockSpec((BB, I, F), lambda g: (g, 0, 0)),
            pl.BlockSpec((I, F), lambda g: (0, 0)),
        ],
        out_specs=pl.BlockSpec((1, I), lambda g: (0, 0)),
        out_shape=jax.ShapeDtypeStruct((1, I), jnp.float32),
        scratch_shapes=[pltpu.VMEM((I, F), jnp.float32)],
    )(out, labels, importance)
    return res[0]
